# Initial kernel scaffold; baseline (speedup 1.0000x reference)
import jax
import jax.numpy as jnp
from jax.experimental import pallas as pl


def kernel(p, x, params):
    raise NotImplementedError("write your pallas kernel here")


# speedup vs baseline: 18.7785x; 18.7785x over previous
import functools
from functools import partial

import jax
import jax.numpy as jnp
import numpy as np
from jax import lax
from jax.experimental import pallas as pl
from jax.experimental.pallas import tpu as pltpu
from jax.experimental.pallas import tpu_sc as plsc

BN_EPS = 1e-5
F32 = jnp.float32


def _fps_body(n_samples, p_ref, out_ref):
    B, _, N = p_ref.shape
    px = p_ref[:, 0, :]
    py = p_ref[:, 1, :]
    pz = p_ref[:, 2, :]
    lane = lax.broadcasted_iota(jnp.int32, (B, N), 1)
    lane_s = lax.broadcasted_iota(jnp.int32, (B, n_samples), 1)

    def body(i, state):
        dist, far, cxa, cya, cza = state
        sel = lane == far
        cx = jnp.sum(jnp.where(sel, px, 0.0), axis=1, keepdims=True)
        cy = jnp.sum(jnp.where(sel, py, 0.0), axis=1, keepdims=True)
        cz = jnp.sum(jnp.where(sel, pz, 0.0), axis=1, keepdims=True)
        here = lane_s == i
        cxa = jnp.where(here, cx, cxa)
        cya = jnp.where(here, cy, cya)
        cza = jnp.where(here, cz, cza)
        d = (px - cx) ** 2 + (py - cy) ** 2 + (pz - cz) ** 2
        dist = jnp.minimum(dist, d)
        m = jnp.max(dist, axis=1, keepdims=True)
        far = jnp.min(jnp.where(dist == m, lane, N), axis=1, keepdims=True)
        return dist, far, cxa, cya, cza

    dist0 = jnp.full((B, N), 1e10, F32)
    far0 = jnp.zeros((B, 1), jnp.int32)
    zs = jnp.zeros((B, n_samples), F32)
    _, _, cxa, cya, cza = lax.fori_loop(0, n_samples, body,
                                        (dist0, far0, zs, zs, zs))
    out_ref[:, 0, :] = cxa
    out_ref[:, 1, :] = cya
    out_ref[:, 2, :] = cza


def _fps(p, n_samples):
    B = p.shape[0]
    return pl.pallas_call(
        partial(_fps_body, n_samples),
        out_shape=jax.ShapeDtypeStruct((B, 3, n_samples), F32),
    )(p)


def _ballq_body(radius, nsample, n_points, c_ref, p_ref, out_ref):
    S = c_ref.shape[1]
    N = n_points
    b = pl.program_id(0)
    sb = pl.program_id(1)
    del sb
    c = c_ref[0]
    cx = c[:, 0:1]
    cy = c[:, 1:2]
    cz = c[:, 2:3]
    px = p_ref[0, 0:1, :]
    py = p_ref[0, 1:2, :]
    pz = p_ref[0, 2:3, :]
    sqd = (cx - px) ** 2 + (cy - py) ** 2 + (cz - pz) ** 2
    mask = (sqd <= radius * radius).astype(F32)
    cs = mask
    sh = 1
    while sh < N:
        cs = cs + jnp.concatenate(
            [jnp.zeros((S, sh), F32), cs[:, : N - sh]], axis=1)
        sh *= 2
    cols = []
    for r in range(nsample):
        cnt = jnp.sum((cs <= float(r)).astype(F32), axis=1, keepdims=True)
        cols.append(cnt)
    idx = jnp.concatenate(cols, axis=1)
    first = idx[:, 0:1]
    first = jnp.where(first >= N, 0.0, first)
    idx = jnp.where(idx >= N, first, idx)
    out_ref[0] = idx.astype(jnp.int32) + b * N


def _ballq(radius, nsample, c, p, s_blk):
    B, S, _ = c.shape
    N = p.shape[2]
    return pl.pallas_call(
        partial(_ballq_body, radius, nsample, N),
        grid=(B, S // s_blk),
        in_specs=[
            pl.BlockSpec((1, s_blk, 3), lambda b, s: (b, s, 0)),
            pl.BlockSpec((1, 3, N), lambda b, s: (b, 0, 0)),
        ],
        out_specs=pl.BlockSpec((1, s_blk, nsample), lambda b, s: (b, s, 0)),
        out_shape=jax.ShapeDtypeStruct((B, S, nsample), jnp.int32),
    )(c, p)


def _sc_gather(table, idx, n_rows, row_w):
    info = plsc.get_sparse_core_info()
    nw = info.num_cores * info.num_subcores
    per_w = n_rows // nw
    chunk = 128
    n_chunks = per_w // chunk
    mesh = plsc.VectorSubcoreMesh(core_axis_name="c", subcore_axis_name="s")

    @functools.partial(
        pl.kernel, mesh=mesh,
        compiler_params=pltpu.CompilerParams(use_tc_tiling_on_sc=False),
        out_type=jax.ShapeDtypeStruct((n_rows, row_w), F32),
        scratch_types=[
            pltpu.VMEM((chunk,), jnp.int32),
            pltpu.VMEM((chunk, row_w), F32),
            pltpu.SemaphoreType.DMA,
        ],
    )
    def k(table_hbm, idx_hbm, out_hbm, idx_v, rows_v, sem):
        wid = lax.axis_index("s") * info.num_cores + lax.axis_index("c")
        base0 = wid * per_w

        def step(i, carry):
            base = base0 + i * chunk
            pltpu.sync_copy(idx_hbm.at[pl.ds(base, chunk)], idx_v)
            pltpu.async_copy(table_hbm.at[idx_v], rows_v, sem).wait()
            pltpu.sync_copy(rows_v, out_hbm.at[pl.ds(base, chunk)])
            return carry

        lax.fori_loop(0, n_chunks, step, 0)

    return k(table, idx)


def _sa_mlp_body(nsample, c_in, ws, c_ref, g_ref, *rest):
    w1, w1c, b1, w2, b2, w3, b3, out_ref = rest
    S = c_ref.shape[1]
    g = g_ref[0]
    c = c_ref[0]
    u = jnp.dot(g, w1[...], preferred_element_type=F32) + b1[...]
    wc = jnp.dot(c, w1c[...], preferred_element_type=F32)
    h1 = u.shape[-1]
    y = u.reshape(S, nsample, h1) - wc[:, None, :]
    y = jnp.maximum(y, 0.0).reshape(S * nsample, h1)
    y = jnp.maximum(jnp.dot(y, w2[...], preferred_element_type=F32)
                    + b2[...], 0.0)
    y = jnp.maximum(jnp.dot(y, w3[...], preferred_element_type=F32)
                    + b3[...], 0.0)
    h3 = y.shape[-1]
    out_ref[0] = jnp.max(y.reshape(S, nsample, h3), axis=1)


def _sa_mlp(g, c, layers, nsample, s_blk):
    B, S, _ = c.shape
    c_pad = g.shape[-1]
    (w1, b1), (w2, b2), (w3, b3) = layers
    w1p = jnp.zeros((c_pad, w1.shape[1]), F32).at[: w1.shape[0]].set(w1)
    w1c = w1[:3]
    h3 = w3.shape[1]
    wspec = lambda a: pl.BlockSpec(a.shape, lambda b, s: tuple(0 for _ in a.shape))
    return pl.pallas_call(
        partial(_sa_mlp_body, nsample, c_pad, None),
        grid=(B, S // s_blk),
        in_specs=[
            pl.BlockSpec((1, s_blk, 3), lambda b, s: (b, s, 0)),
            pl.BlockSpec((1, s_blk * nsample, c_pad), lambda b, s: (b, s, 0)),
            wspec(w1p), wspec(w1c), wspec(b1),
            wspec(w2), wspec(b2), wspec(w3), wspec(b3),
        ],
        out_specs=pl.BlockSpec((1, s_blk, h3), lambda b, s: (b, s, 0)),
        out_shape=jax.ShapeDtypeStruct((B, S, h3), F32),
    )(c, g, w1p, w1c, b1, w2, b2, w3, b3)


def _sa3_head_body(x_ref, *rest):
    (w1, b1, w2, b2, w3, b3,
     wh1, bh1, wh2, bh2, wfc, bfc, out_ref) = rest
    g = x_ref[0]
    y = jnp.maximum(jnp.dot(g, w1[...], preferred_element_type=F32)
                    + b1[...], 0.0)
    y = jnp.maximum(jnp.dot(y, w2[...], preferred_element_type=F32)
                    + b2[...], 0.0)
    y = jnp.maximum(jnp.dot(y, w3[...], preferred_element_type=F32)
                    + b3[...], 0.0)
    h = jnp.max(y, axis=0, keepdims=True)
    h = jnp.maximum(jnp.dot(h, wh1[...], preferred_element_type=F32)
                    + bh1[...], 0.0)
    h = jnp.maximum(jnp.dot(h, wh2[...], preferred_element_type=F32)
                    + bh2[...], 0.0)
    out_ref[0] = jnp.dot(h, wfc[...], preferred_element_type=F32) + bfc[...]


def _sa3_head(x3, layers, head):
    B = x3.shape[0]
    (w1, b1), (w2, b2), (w3, b3) = layers
    wh1, bh1, wh2, bh2, wfc, bfc = head
    args = [w1, b1, w2, b2, w3, b3, wh1, bh1, wh2, bh2, wfc, bfc]
    wspec = lambda a: pl.BlockSpec(a.shape, lambda b: tuple(0 for _ in a.shape))
    out = pl.pallas_call(
        _sa3_head_body,
        grid=(B,),
        in_specs=[pl.BlockSpec((1,) + x3.shape[1:], lambda b: (b, 0, 0))]
        + [wspec(a) for a in args],
        out_specs=pl.BlockSpec((1, 1, 40), lambda b: (b, 0, 0)),
        out_shape=jax.ShapeDtypeStruct((B, 1, 40), F32),
    )(x3, *args)
    return out.reshape(B, 40)


def _scaled_layers(layers):
    s = 1.0 / np.sqrt(1.0 + BN_EPS)
    return [(jnp.asarray((l['W'] * (l['gamma'] * s)[:, None]).T),
             jnp.asarray(l['beta'])[None, :]) for l in layers]


def kernel(p, x, params):
    B, _, N = p.shape
    sa1 = _scaled_layers(params['sa1'])
    sa2 = _scaled_layers(params['sa2'])
    sa3 = _scaled_layers(params['sa3'])
    hd = params['head']
    s = 1.0 / np.sqrt(1.0 + BN_EPS)
    head = (jnp.asarray((hd['W1'] * (hd['g1'] * s)[:, None]).T),
            hd['b1'][None, :],
            jnp.asarray((hd['W2'] * (hd['g2'] * s)[:, None]).T),
            hd['b2'][None, :],
            jnp.asarray(hd['Wfc'].T), hd['bfc'][None, :])

    c1p = _fps(p, 512)
    c1 = jnp.transpose(c1p, (0, 2, 1))
    idx1 = _ballq(0.2, 32, c1, p, 128)
    tab1 = jnp.concatenate(
        [jnp.transpose(p, (0, 2, 1)), jnp.transpose(x, (0, 2, 1)),
         jnp.zeros((B, N, 2), F32)], axis=-1).reshape(B * N, 8)
    g1 = _sc_gather(tab1, idx1.reshape(-1), B * 512 * 32, 8)
    g1 = g1.reshape(B, 512 * 32, 8)
    f1 = _sa_mlp(g1, c1, sa1, 32, 128)

    c2p = _fps(c1p, 128)
    c2 = jnp.transpose(c2p, (0, 2, 1))
    idx2 = _ballq(0.4, 64, c2, c1p, 128)
    tab2 = jnp.concatenate(
        [c1, f1, jnp.zeros((B, 512, 13), F32)], axis=-1).reshape(B * 512, 144)
    g2 = _sc_gather(tab2, idx2.reshape(-1), B * 128 * 64, 144)
    g2 = g2.reshape(B, 128 * 64, 144)
    f2 = _sa_mlp(g2, c2, sa2, 64, 64)

    x3 = jnp.concatenate([c2, f2], axis=-1)
    return _sa3_head(x3, sa3, head)
